# baseline (speedup 1.0000x reference)
import functools

import jax
import jax.numpy as jnp
from jax import lax
from jax.experimental import pallas as pl
from jax.experimental.pallas import tpu as pltpu
from jax.experimental.pallas import tpu_sc as plsc

N = 10000
E = 320000
H = 128
DEPTH = 4

NW = 32
CHUNK = 128
NCHUNK = 80
TPT = NCHUNK * CHUNK
E_PAD = NW * TPT
NACC = 10240
STRIPE = NACC // 16
BE = 2048
F32 = jnp.float32
BF16 = jnp.bfloat16
HP = H // 2


def _silu(x):
    return x * jax.nn.sigmoid(x)


def _sc_mesh():
    return plsc.VectorSubcoreMesh(core_axis_name="c", subcore_axis_name="s")


NB = 5
NQ = NCHUNK // NB


def _gather_body(a_hbm, b_hbm, src_hbm, dst_hbm, gs_hbm, gd_hbm,
                 sidx, didx, *rest):
    bufs = rest[:NB]
    semg = rest[NB:2 * NB]
    semw = rest[2 * NB:3 * NB]
    c = lax.axis_index("c")
    s = lax.axis_index("s")
    wid = c * 16 + s
    pltpu.sync_copy(src_hbm.at[wid], sidx)
    pltpu.sync_copy(dst_hbm.at[wid], didx)
    base = wid * TPT

    for tab, idx, out_hbm in ((a_hbm, sidx, gs_hbm), (b_hbm, didx, gd_hbm)):

        def fire_g(k, i):
            pltpu.async_copy(tab.at[idx.at[k]], bufs[i], semg[i])

        def wait_g(k, i):
            pltpu.make_async_copy(tab.at[idx.at[k]], bufs[i], semg[i]).wait()

        def wb(k):
            return out_hbm.at[pl.ds(base + k * CHUNK, CHUNK)]

        for i in range(NB):
            fire_g(i, i)

        def step(q, carry):
            k0 = NB * q
            for i in range(NB):
                wait_g(k0 + i, i)
                pltpu.async_copy(bufs[i], wb(k0 + i), semw[i])

            @pl.when(q < NQ - 1)
            def _tail():
                for i in range(NB):
                    pltpu.make_async_copy(bufs[i], wb(k0 + i), semw[i]).wait()
                    fire_g(k0 + NB + i, i)

            return carry

        lax.fori_loop(0, NQ, step, None)
        for i in range(NB):
            pltpu.make_async_copy(bufs[i], wb(NCHUNK - NB + i), semw[i]).wait()


def _sc_gather(a, b, src3, dst3):
    kern = functools.partial(
        pl.kernel,
        mesh=_sc_mesh(),
        out_type=[jax.ShapeDtypeStruct((E_PAD, H), F32),
                  jax.ShapeDtypeStruct((E_PAD, H), F32)],
        scratch_types=(
            [pltpu.VMEM((NCHUNK, CHUNK), jnp.int32)] * 2
            + [pltpu.VMEM((CHUNK, H), F32)] * NB
            + [pltpu.SemaphoreType.DMA] * (2 * NB)
        ),
    )(_gather_body)
    return kern(a, b, src3, dst3)


def _scatter_body(mw_hbm, dst_hbm, part_hbm, didx, mv, mv1, acc, semr0, semr1):
    c = lax.axis_index("c")
    s = lax.axis_index("s")
    wid = c * 16 + s

    def zrow(r, _):
        for cc in range(H // 16):
            mv[r, pl.ds(cc * 16, 16)] = jnp.zeros((16,), F32)
        return _

    lax.fori_loop(0, CHUNK, zrow, None)
    for q in range(STRIPE // CHUNK):
        pltpu.sync_copy(mv, acc.at[pl.ds(s * STRIPE + q * CHUNK, CHUNK)])
    plsc.subcore_barrier()

    pltpu.sync_copy(dst_hbm.at[wid], didx)

    def rd_slice(k):
        return mw_hbm.at[pl.ds(wid * TPT + k * CHUNK, CHUNK)]

    pltpu.async_copy(rd_slice(0), mv, semr0)
    pltpu.async_copy(rd_slice(1), mv1, semr1)

    def pair(j, carry):
        k0 = 2 * j
        k1 = k0 + 1
        pltpu.make_async_copy(rd_slice(k0), mv, semr0).wait()
        pltpu.sync_copy(mv, acc.at[didx.at[k0]], add=True)

        @pl.when(j < NCHUNK // 2 - 1)
        def _t0():
            pltpu.async_copy(rd_slice(k0 + 2), mv, semr0)

        pltpu.make_async_copy(rd_slice(k1), mv1, semr1).wait()
        pltpu.sync_copy(mv1, acc.at[didx.at[k1]], add=True)

        @pl.when(j < NCHUNK // 2 - 1)
        def _t1():
            pltpu.async_copy(rd_slice(k1 + 2), mv1, semr1)

        return carry

    lax.fori_loop(0, NCHUNK // 2, pair, None)
    plsc.subcore_barrier()
    pltpu.sync_copy(acc.at[pl.ds(s * STRIPE, STRIPE)],
                    part_hbm.at[c, pl.ds(s * STRIPE, STRIPE)])


def _sc_scatter(mw, dst3):
    kern = functools.partial(
        pl.kernel,
        mesh=_sc_mesh(),
        out_type=jax.ShapeDtypeStruct((2, NACC, H), F32),
        scratch_types=[
            pltpu.VMEM((NCHUNK, CHUNK), jnp.int32),
            pltpu.VMEM((CHUNK, H), F32),
            pltpu.VMEM((CHUNK, H), F32),
            pltpu.VMEM_SHARED((NACC, H), F32),
            pltpu.SemaphoreType.DMA,
            pltpu.SemaphoreType.DMA,
        ],
    )(_scatter_body)
    return kern(mw, dst3)


def _edge0_body(ed_ref, ne_ref, w1s_ref, b1_ref, eiw_ref, eib_ref,
                w1d_ref, w2_ref, b2_ref, sew_ref, seb_ref, mw_ref, dn_ref):
    row = jnp.dot(ne_ref[...], w1s_ref[...], preferred_element_type=F32)
    row = row + b1_ref[...]
    d = _silu(_silu(ed_ref[...] * eiw_ref[...] + eib_ref[...]))
    pre = row + jnp.dot(d, w1d_ref[...], preferred_element_type=F32)
    m1 = _silu(pre)
    m = _silu(jnp.dot(m1, w2_ref[...], preferred_element_type=F32) + b2_ref[...])
    w = jax.nn.sigmoid(jnp.sum(m * sew_ref[...], axis=1, keepdims=True)
                       + seb_ref[...])
    mw_ref[...] = m * w
    dn_ref[...] = d + m


def _edge_body(gs_ref, gd_ref, d_ref, w1d_ref, w2_ref, b2_ref, sew_ref,
               seb_ref, mw_ref, *rest):
    d = d_ref[...]
    pre = (gs_ref[...].astype(F32) + gd_ref[...].astype(F32)
           + jnp.dot(d, w1d_ref[...], preferred_element_type=F32))
    m1 = _silu(pre)
    m = _silu(jnp.dot(m1, w2_ref[...], preferred_element_type=F32) + b2_ref[...])
    w = jax.nn.sigmoid(jnp.sum(m * sew_ref[...], axis=1, keepdims=True)
                       + seb_ref[...])
    mw_ref[...] = m * w
    if rest:
        rest[0][...] = d + m


_EDGE_GRID = E_PAD // BE


def _ebs():
    return pl.BlockSpec((BE, H), lambda i: (i, 0))


def _wbs(shape):
    return pl.BlockSpec(shape, lambda i: (0, 0))


def _tc_edge0(edge_d, node_emb2, w1s, b1, eiw, eib, w1d, w2, b2, sew, seb):
    return pl.pallas_call(
        _edge0_body,
        grid=(_EDGE_GRID,),
        in_specs=[
            pl.BlockSpec((BE, 1), lambda i: (i, 0)),
            _wbs((1, H)), _wbs((H, H)), _wbs((1, H)), _wbs((1, H)),
            _wbs((1, H)), _wbs((H, H)), _wbs((H, H)), _wbs((1, H)),
            _wbs((1, H)), _wbs((1, 1)),
        ],
        out_specs=[_ebs(), _ebs()],
        out_shape=[jax.ShapeDtypeStruct((E_PAD, H), F32),
                   jax.ShapeDtypeStruct((E_PAD, H), F32)],
    )(edge_d, node_emb2, w1s, b1, eiw, eib, w1d, w2, b2, sew, seb)


def _tc_edge(gs, gd, d, w1d, w2, b2, sew, seb, want_dnew):
    n_out = 2 if want_dnew else 1
    out = pl.pallas_call(
        _edge_body,
        grid=(_EDGE_GRID,),
        in_specs=[
            _ebs(), _ebs(), _ebs(),
            _wbs((H, H)), _wbs((H, H)), _wbs((1, H)), _wbs((1, H)),
            _wbs((1, 1)),
        ],
        out_specs=[_ebs()] * n_out,
        out_shape=[jax.ShapeDtypeStruct((E_PAD, H), F32)] * n_out,
    )(gs, gd, d, w1d, w2, b2, sew, seb)
    return out


def _update_body(part_ref, feat_ref, uw1_ref, ub1_ref, uw2_ref, ub2_ref,
                 w1a_ref, w1b_ref, b1n_ref, fn_ref, a_ref, b_ref, *, first):
    part = part_ref[...]
    if first:
        feat = jnp.broadcast_to(feat_ref[...], (NACC, H))
    else:
        feat = feat_ref[...]
    s = part[0] + part[1] + feat
    h = _silu(jnp.dot(s, uw1_ref[...], preferred_element_type=F32) + ub1_ref[...])
    h = jnp.dot(h, uw2_ref[...], preferred_element_type=F32) + ub2_ref[...]
    fnew = feat + h
    fn_ref[...] = fnew
    a_ref[...] = jnp.dot(fnew, w1a_ref[...],
                         preferred_element_type=F32) + b1n_ref[...]
    b_ref[...] = jnp.dot(fnew, w1b_ref[...], preferred_element_type=F32)


def _tc_update(part, feat, uw1, ub1, uw2, ub2, w1a, w1b, b1n, first):
    return pl.pallas_call(
        functools.partial(_update_body, first=first),
        out_shape=[jax.ShapeDtypeStruct((NACC, H), F32)] * 3,
    )(part, feat, uw1, ub1, uw2, ub2, w1a, w1b, b1n)


def _final_body(part_ref, feat_ref, uw1_ref, ub1_ref, uw2_ref, ub2_ref,
                ow1_ref, ob1_ref, ow2_ref, ob2_ref, out_ref):
    part = part_ref[...]
    feat = feat_ref[...]
    s = part[0] + part[1] + feat
    h = _silu(jnp.dot(s, uw1_ref[...], preferred_element_type=F32) + ub1_ref[...])
    h = jnp.dot(h, uw2_ref[...], preferred_element_type=F32) + ub2_ref[...]
    f4 = feat + h
    o = _silu(jnp.dot(f4, ow1_ref[...], preferred_element_type=F32) + ob1_ref[...])
    o = jnp.dot(o, ow2_ref[...], preferred_element_type=F32) + ob2_ref[...]
    o = o[:N]
    rsum = jnp.sum(o, axis=0)
    rmean = rsum * (1.0 / N)
    rmax = jnp.max(o, axis=0)
    out_ref[...] = jnp.concatenate([rsum, rmean, rmax], axis=0).reshape(1, 3 * H)


def _tc_final(part, feat, uw1, ub1, uw2, ub2, ow1, ob1, ow2, ob2):
    return pl.pallas_call(
        _final_body,
        out_shape=jax.ShapeDtypeStruct((1, 3 * H), F32),
    )(part, feat, uw1, ub1, uw2, ub2, ow1, ob1, ow2, ob2)


def kernel(edge_index, edge_d, node_emb, ei_W, ei_b, msg_W1, msg_b1, msg_W2,
           msg_b2, se_W, se_b, up_W1, up_b1, up_W2, up_b2, out_W1, out_b1,
           out_W2, out_b2):
    npad = E_PAD - E
    src = jnp.concatenate([edge_index[0], jnp.zeros((npad,), jnp.int32)])
    dstp = jnp.concatenate(
        [edge_index[1], N + (jnp.arange(npad, dtype=jnp.int32) % 16)])
    src3 = src.reshape(NW, NCHUNK, CHUNK)
    dst3 = dstp.reshape(NW, NCHUNK, CHUNK)
    edge_dp = jnp.concatenate([edge_d, jnp.zeros((npad, 1), F32)], axis=0)

    ne2 = node_emb.reshape(1, H)
    eiw = ei_W.reshape(1, H)
    eib = ei_b.reshape(1, H)

    def lw(l):
        return dict(
            w1a=msg_W1[l, :H], w1b=msg_W1[l, H:2 * H], w1d=msg_W1[l, 2 * H:],
            b1=msg_b1[l].reshape(1, H), w2=msg_W2[l],
            b2=msg_b2[l].reshape(1, H), sew=se_W[l, :, 0].reshape(1, H),
            seb=se_b[l].reshape(1, 1), uw1=up_W1[l],
            ub1=up_b1[l].reshape(1, H), uw2=up_W2[l],
            ub2=up_b2[l].reshape(1, H),
        )

    w = [lw(l) for l in range(DEPTH)]

    w1s0 = w[0]["w1a"] + w[0]["w1b"]
    mw, d = _tc_edge0(edge_dp, ne2, w1s0, w[0]["b1"], eiw, eib,
                      w[0]["w1d"], w[0]["w2"], w[0]["b2"],
                      w[0]["sew"], w[0]["seb"])
    part = _sc_scatter(mw, dst3)
    feat, a, b = _tc_update(part, ne2, w[0]["uw1"], w[0]["ub1"],
                            w[0]["uw2"], w[0]["ub2"],
                            w[1]["w1a"], w[1]["w1b"], w[1]["b1"], first=True)

    for l in range(1, DEPTH):
        gs, gd = _sc_gather(a, b, src3, dst3)
        last = l == DEPTH - 1
        out = _tc_edge(gs, gd, d, w[l]["w1d"], w[l]["w2"],
                       w[l]["b2"], w[l]["sew"], w[l]["seb"],
                       want_dnew=not last)
        if last:
            mw = out[0]
        else:
            mw, d = out
        part = _sc_scatter(mw, dst3)
        if not last:
            nxt = w[l + 1]
            feat, a, b = _tc_update(part, feat, w[l]["uw1"], w[l]["ub1"],
                                    w[l]["uw2"], w[l]["ub2"],
                                    nxt["w1a"], nxt["w1b"], nxt["b1"],
                                    first=False)

    wl = w[DEPTH - 1]
    return _tc_final(part, feat, wl["uw1"], wl["ub1"], wl["uw2"], wl["ub2"],
                     out_W1, out_b1.reshape(1, H), out_W2,
                     out_b2.reshape(1, H))


# speedup vs baseline: 1.1236x; 1.1236x over previous
import functools

import jax
import jax.numpy as jnp
from jax import lax
from jax.experimental import pallas as pl
from jax.experimental.pallas import tpu as pltpu
from jax.experimental.pallas import tpu_sc as plsc

N = 10000
E = 320000
H = 128
DEPTH = 4

NW = 32
CHUNK = 128
NCHUNK = 80
TPT = NCHUNK * CHUNK
E_PAD = NW * TPT
NACC = 10240
STRIPE = NACC // 16
BE = 2048
F32 = jnp.float32


def _silu(x):
    return x * jax.nn.sigmoid(x)


def _sc_mesh():
    return plsc.VectorSubcoreMesh(core_axis_name="c", subcore_axis_name="s")


def _gather_body(a_hbm, b_hbm, src_hbm, dst_hbm, g_hbm,
                 sidx, didx, a0, b0, a1, b1, sa0, sb0, sa1, sb1):
    c = lax.axis_index("c")
    s = lax.axis_index("s")
    wid = c * 16 + s
    pltpu.sync_copy(src_hbm.at[wid], sidx)
    pltpu.sync_copy(dst_hbm.at[wid], didx)
    base = wid * TPT

    def fire(k, abuf, bbuf, sa, sb):
        pltpu.async_copy(a_hbm.at[sidx.at[k]], abuf, sa)
        pltpu.async_copy(b_hbm.at[didx.at[k]], bbuf, sb)

    def addbuf(abuf, bbuf):
        def row(r, carry):
            for cc in range(H // 16):
                sl = pl.ds(cc * 16, 16)
                abuf[r, sl] = abuf[r, sl] + bbuf[r, sl]
            return carry

        lax.fori_loop(0, CHUNK, row, None)

    fire(0, a0, b0, sa0, sb0)
    fire(1, a1, b1, sa1, sb1)

    def pair(j, carry):
        k0 = 2 * j
        k1 = k0 + 1
        pltpu.make_async_copy(a_hbm.at[sidx.at[k0]], a0, sa0).wait()
        pltpu.make_async_copy(b_hbm.at[didx.at[k0]], b0, sb0).wait()
        addbuf(a0, b0)
        pltpu.sync_copy(a0, g_hbm.at[pl.ds(base + k0 * CHUNK, CHUNK)])

        @pl.when(j < NCHUNK // 2 - 1)
        def _t0():
            fire(k0 + 2, a0, b0, sa0, sb0)

        pltpu.make_async_copy(a_hbm.at[sidx.at[k1]], a1, sa1).wait()
        pltpu.make_async_copy(b_hbm.at[didx.at[k1]], b1, sb1).wait()
        addbuf(a1, b1)
        pltpu.sync_copy(a1, g_hbm.at[pl.ds(base + k1 * CHUNK, CHUNK)])

        @pl.when(j < NCHUNK // 2 - 1)
        def _t1():
            fire(k1 + 2, a1, b1, sa1, sb1)

        return carry

    lax.fori_loop(0, NCHUNK // 2, pair, None)


def _sc_gather(a, b, src3, dst3):
    kern = functools.partial(
        pl.kernel,
        mesh=_sc_mesh(),
        out_type=jax.ShapeDtypeStruct((E_PAD, H), F32),
        scratch_types=(
            [pltpu.VMEM((NCHUNK, CHUNK), jnp.int32)] * 2
            + [pltpu.VMEM((CHUNK, H), F32)] * 4
            + [pltpu.SemaphoreType.DMA] * 4
        ),
    )(_gather_body)
    return kern(a, b, src3, dst3)


def _scatter_body(mw_hbm, dst_hbm, part_hbm, didx, mv, mv1, acc, semr0, semr1):
    c = lax.axis_index("c")
    s = lax.axis_index("s")
    wid = c * 16 + s

    def zrow(r, _):
        for cc in range(H // 16):
            mv[r, pl.ds(cc * 16, 16)] = jnp.zeros((16,), F32)
        return _

    lax.fori_loop(0, CHUNK, zrow, None)
    for q in range(STRIPE // CHUNK):
        pltpu.sync_copy(mv, acc.at[pl.ds(s * STRIPE + q * CHUNK, CHUNK)])
    plsc.subcore_barrier()

    pltpu.sync_copy(dst_hbm.at[wid], didx)

    def rd_slice(k):
        return mw_hbm.at[pl.ds(wid * TPT + k * CHUNK, CHUNK)]

    pltpu.async_copy(rd_slice(0), mv, semr0)
    pltpu.async_copy(rd_slice(1), mv1, semr1)

    def pair(j, carry):
        k0 = 2 * j
        k1 = k0 + 1
        pltpu.make_async_copy(rd_slice(k0), mv, semr0).wait()
        pltpu.sync_copy(mv, acc.at[didx.at[k0]], add=True)

        @pl.when(j < NCHUNK // 2 - 1)
        def _t0():
            pltpu.async_copy(rd_slice(k0 + 2), mv, semr0)

        pltpu.make_async_copy(rd_slice(k1), mv1, semr1).wait()
        pltpu.sync_copy(mv1, acc.at[didx.at[k1]], add=True)

        @pl.when(j < NCHUNK // 2 - 1)
        def _t1():
            pltpu.async_copy(rd_slice(k1 + 2), mv1, semr1)

        return carry

    lax.fori_loop(0, NCHUNK // 2, pair, None)
    plsc.subcore_barrier()
    pltpu.sync_copy(acc.at[pl.ds(s * STRIPE, STRIPE)],
                    part_hbm.at[c, pl.ds(s * STRIPE, STRIPE)])


def _sc_scatter(mw, dst3):
    kern = functools.partial(
        pl.kernel,
        mesh=_sc_mesh(),
        out_type=jax.ShapeDtypeStruct((2, NACC, H), F32),
        scratch_types=[
            pltpu.VMEM((NCHUNK, CHUNK), jnp.int32),
            pltpu.VMEM((CHUNK, H), F32),
            pltpu.VMEM((CHUNK, H), F32),
            pltpu.VMEM_SHARED((NACC, H), F32),
            pltpu.SemaphoreType.DMA,
            pltpu.SemaphoreType.DMA,
        ],
    )(_scatter_body)
    return kern(mw, dst3)


def _edge0_body(ed_ref, ne_ref, w1s_ref, b1_ref, eiw_ref, eib_ref,
                w1d_ref, w2_ref, b2_ref, sew_ref, seb_ref, mw_ref, dn_ref):
    row = jnp.dot(ne_ref[...], w1s_ref[...], preferred_element_type=F32)
    row = row + b1_ref[...]
    d = _silu(_silu(ed_ref[...] * eiw_ref[...] + eib_ref[...]))
    pre = row + jnp.dot(d, w1d_ref[...], preferred_element_type=F32)
    m1 = _silu(pre)
    m = _silu(jnp.dot(m1, w2_ref[...], preferred_element_type=F32) + b2_ref[...])
    w = jax.nn.sigmoid(jnp.sum(m * sew_ref[...], axis=1, keepdims=True)
                       + seb_ref[...])
    mw_ref[...] = m * w
    dn_ref[...] = d + m


def _edge_body(g_ref, d_ref, w1d_ref, w2_ref, b2_ref, sew_ref,
               seb_ref, mw_ref, *rest):
    d = d_ref[...]
    pre = g_ref[...] + jnp.dot(d, w1d_ref[...], preferred_element_type=F32)
    m1 = _silu(pre)
    m = _silu(jnp.dot(m1, w2_ref[...], preferred_element_type=F32) + b2_ref[...])
    w = jax.nn.sigmoid(jnp.sum(m * sew_ref[...], axis=1, keepdims=True)
                       + seb_ref[...])
    mw_ref[...] = m * w
    if rest:
        rest[0][...] = d + m


_EDGE_GRID = E_PAD // BE


def _ebs():
    return pl.BlockSpec((BE, H), lambda i: (i, 0))


def _wbs(shape):
    return pl.BlockSpec(shape, lambda i: (0, 0))


def _tc_edge0(edge_d, node_emb2, w1s, b1, eiw, eib, w1d, w2, b2, sew, seb):
    return pl.pallas_call(
        _edge0_body,
        grid=(_EDGE_GRID,),
        in_specs=[
            pl.BlockSpec((BE, 1), lambda i: (i, 0)),
            _wbs((1, H)), _wbs((H, H)), _wbs((1, H)), _wbs((1, H)),
            _wbs((1, H)), _wbs((H, H)), _wbs((H, H)), _wbs((1, H)),
            _wbs((1, H)), _wbs((1, 1)),
        ],
        out_specs=[_ebs(), _ebs()],
        out_shape=[jax.ShapeDtypeStruct((E_PAD, H), F32),
                   jax.ShapeDtypeStruct((E_PAD, H), F32)],
    )(edge_d, node_emb2, w1s, b1, eiw, eib, w1d, w2, b2, sew, seb)


def _tc_edge(g, d, w1d, w2, b2, sew, seb, want_dnew):
    n_out = 2 if want_dnew else 1
    out = pl.pallas_call(
        _edge_body,
        grid=(_EDGE_GRID,),
        in_specs=[
            _ebs(), _ebs(),
            _wbs((H, H)), _wbs((H, H)), _wbs((1, H)), _wbs((1, H)),
            _wbs((1, 1)),
        ],
        out_specs=[_ebs()] * n_out,
        out_shape=[jax.ShapeDtypeStruct((E_PAD, H), F32)] * n_out,
    )(g, d, w1d, w2, b2, sew, seb)
    return out


def _update_body(part_ref, feat_ref, uw1_ref, ub1_ref, uw2_ref, ub2_ref,
                 w1a_ref, w1b_ref, b1n_ref, fn_ref, a_ref, b_ref, *, first):
    part = part_ref[...]
    if first:
        feat = jnp.broadcast_to(feat_ref[...], (NACC, H))
    else:
        feat = feat_ref[...]
    s = part[0] + part[1] + feat
    h = _silu(jnp.dot(s, uw1_ref[...], preferred_element_type=F32) + ub1_ref[...])
    h = jnp.dot(h, uw2_ref[...], preferred_element_type=F32) + ub2_ref[...]
    fnew = feat + h
    fn_ref[...] = fnew
    a_ref[...] = jnp.dot(fnew, w1a_ref[...],
                         preferred_element_type=F32) + b1n_ref[...]
    b_ref[...] = jnp.dot(fnew, w1b_ref[...], preferred_element_type=F32)


def _tc_update(part, feat, uw1, ub1, uw2, ub2, w1a, w1b, b1n, first):
    return pl.pallas_call(
        functools.partial(_update_body, first=first),
        out_shape=[jax.ShapeDtypeStruct((NACC, H), F32)] * 3,
    )(part, feat, uw1, ub1, uw2, ub2, w1a, w1b, b1n)


def _final_body(part_ref, feat_ref, uw1_ref, ub1_ref, uw2_ref, ub2_ref,
                ow1_ref, ob1_ref, ow2_ref, ob2_ref, out_ref):
    part = part_ref[...]
    feat = feat_ref[...]
    s = part[0] + part[1] + feat
    h = _silu(jnp.dot(s, uw1_ref[...], preferred_element_type=F32) + ub1_ref[...])
    h = jnp.dot(h, uw2_ref[...], preferred_element_type=F32) + ub2_ref[...]
    f4 = feat + h
    o = _silu(jnp.dot(f4, ow1_ref[...], preferred_element_type=F32) + ob1_ref[...])
    o = jnp.dot(o, ow2_ref[...], preferred_element_type=F32) + ob2_ref[...]
    o = o[:N]
    rsum = jnp.sum(o, axis=0)
    rmean = rsum * (1.0 / N)
    rmax = jnp.max(o, axis=0)
    out_ref[...] = jnp.concatenate([rsum, rmean, rmax], axis=0).reshape(1, 3 * H)


def _tc_final(part, feat, uw1, ub1, uw2, ub2, ow1, ob1, ow2, ob2):
    return pl.pallas_call(
        _final_body,
        out_shape=jax.ShapeDtypeStruct((1, 3 * H), F32),
    )(part, feat, uw1, ub1, uw2, ub2, ow1, ob1, ow2, ob2)


def kernel(edge_index, edge_d, node_emb, ei_W, ei_b, msg_W1, msg_b1, msg_W2,
           msg_b2, se_W, se_b, up_W1, up_b1, up_W2, up_b2, out_W1, out_b1,
           out_W2, out_b2):
    npad = E_PAD - E
    src = jnp.concatenate([edge_index[0], jnp.zeros((npad,), jnp.int32)])
    dstp = jnp.concatenate(
        [edge_index[1], N + (jnp.arange(npad, dtype=jnp.int32) % 16)])
    src3 = src.reshape(NW, NCHUNK, CHUNK)
    dst3 = dstp.reshape(NW, NCHUNK, CHUNK)
    edge_dp = jnp.concatenate([edge_d, jnp.zeros((npad, 1), F32)], axis=0)

    ne2 = node_emb.reshape(1, H)
    eiw = ei_W.reshape(1, H)
    eib = ei_b.reshape(1, H)

    def lw(l):
        return dict(
            w1a=msg_W1[l, :H], w1b=msg_W1[l, H:2 * H], w1d=msg_W1[l, 2 * H:],
            b1=msg_b1[l].reshape(1, H), w2=msg_W2[l],
            b2=msg_b2[l].reshape(1, H), sew=se_W[l, :, 0].reshape(1, H),
            seb=se_b[l].reshape(1, 1), uw1=up_W1[l],
            ub1=up_b1[l].reshape(1, H), uw2=up_W2[l],
            ub2=up_b2[l].reshape(1, H),
        )

    w = [lw(l) for l in range(DEPTH)]

    w1s0 = w[0]["w1a"] + w[0]["w1b"]
    mw, d = _tc_edge0(edge_dp, ne2, w1s0, w[0]["b1"], eiw, eib,
                      w[0]["w1d"], w[0]["w2"], w[0]["b2"],
                      w[0]["sew"], w[0]["seb"])
    part = _sc_scatter(mw, dst3)
    feat, a, b = _tc_update(part, ne2, w[0]["uw1"], w[0]["ub1"],
                            w[0]["uw2"], w[0]["ub2"],
                            w[1]["w1a"], w[1]["w1b"], w[1]["b1"], first=True)

    for l in range(1, DEPTH):
        g = _sc_gather(a, b, src3, dst3)
        last = l == DEPTH - 1
        out = _tc_edge(g, d, w[l]["w1d"], w[l]["w2"],
                       w[l]["b2"], w[l]["sew"], w[l]["seb"],
                       want_dnew=not last)
        if last:
            mw = out[0]
        else:
            mw, d = out
        part = _sc_scatter(mw, dst3)
        if not last:
            nxt = w[l + 1]
            feat, a, b = _tc_update(part, feat, w[l]["uw1"], w[l]["ub1"],
                                    w[l]["uw2"], w[l]["ub2"],
                                    nxt["w1a"], nxt["w1b"], nxt["b1"],
                                    first=False)

    wl = w[DEPTH - 1]
    return _tc_final(part, feat, wl["uw1"], wl["ub1"], wl["uw2"], wl["ub2"],
                     out_W1, out_b1.reshape(1, H), out_W2,
                     out_b2.reshape(1, H))
